# baseline (speedup 1.0000x reference)
import functools

import jax
import jax.numpy as jnp
from jax import lax
from jax.experimental import pallas as pl
from jax.experimental.pallas import tpu as pltpu
from jax.experimental.pallas import tpu_sc as plsc

MAX_LEN = 200
D = 64
BATCH = 4096
SEQ = 200

NC = 2
NS = 16
NW = NC * NS
SEQ_PER_W = BATCH // NW
CHUNK_SEQ = 2
ROWS_PER_CHUNK = CHUNK_SEQ * SEQ
N_CHUNKS = SEQ_PER_W // CHUNK_SEQ
G = 100
K = ROWS_PER_CHUNK // G
XROWS = BATCH * SEQ // G


def _make_kernel():
    mesh = plsc.VectorSubcoreMesh(core_axis_name="c", subcore_axis_name="s")

    @functools.partial(
        pl.kernel,
        mesh=mesh,
        out_type=jax.ShapeDtypeStruct((BATCH * SEQ, D), jnp.float32),
        scratch_types=[
            pltpu.VMEM((K, G), jnp.int32),
            pltpu.VMEM((ROWS_PER_CHUNK, D), jnp.float32),
            pltpu.VMEM((MAX_LEN, D), jnp.float32),
            pltpu.SemaphoreType.DMA,
        ],
        compiler_params=pltpu.CompilerParams(use_tc_tiling_on_sc=False),
    )
    def emb_kernel(x_hbm, tok_hbm, pos_hbm, out_hbm, idx_v, rows_v, pos_v, sem):
        wid = lax.axis_index("s") * NC + lax.axis_index("c")
        pltpu.sync_copy(pos_hbm, pos_v)
        base_xrow = wid * (SEQ_PER_W * SEQ // G)

        def chunk_body(g, carry):
            xrow0 = base_xrow + g * K
            pltpu.sync_copy(x_hbm.at[pl.ds(xrow0, K)], idx_v)
            copies = [
                pltpu.async_copy(
                    tok_hbm.at[idx_v.at[j]], rows_v.at[pl.ds(j * G, G)], sem
                )
                for j in range(K)
            ]
            for cp in copies:
                cp.wait()

            def add_body(r, c2):
                for j in range(D // 16):
                    pv = pos_v[r, pl.ds(j * 16, 16)]
                    for s in range(CHUNK_SEQ):
                        plsc.addupdate(
                            rows_v.at[s * SEQ + r, pl.ds(j * 16, 16)], pv
                        )
                return c2

            lax.fori_loop(0, SEQ, add_body, 0)

            row0 = wid * (SEQ_PER_W * SEQ) + g * ROWS_PER_CHUNK
            pltpu.sync_copy(rows_v, out_hbm.at[pl.ds(row0, ROWS_PER_CHUNK)])
            return carry

        lax.fori_loop(0, N_CHUNKS, chunk_body, 0)

    return emb_kernel


_emb = _make_kernel()


@jax.jit
def kernel(x, token_table, pos_table):
    xf = x.astype(jnp.int32).reshape(XROWS, G)
    out = _emb(xf, token_table, pos_table)
    return out.reshape(BATCH, SEQ, D)


# speedup vs baseline: 1.3695x; 1.3695x over previous
import functools

import jax
import jax.numpy as jnp
from jax import lax
from jax.experimental import pallas as pl
from jax.experimental.pallas import tpu as pltpu
from jax.experimental.pallas import tpu_sc as plsc

MAX_LEN = 200
D = 64
DPAD = 128
BATCH = 4096
SEQ = 200

NC = 2
NS = 16
NW = NC * NS
BPW = BATCH // NW


def _make_kernel():
    mesh = plsc.VectorSubcoreMesh(core_axis_name="c", subcore_axis_name="s")

    @functools.partial(
        pl.kernel,
        mesh=mesh,
        out_type=jax.ShapeDtypeStruct((BATCH, SEQ, DPAD), jnp.float32),
        scratch_types=[
            pltpu.VMEM((2, BPW), jnp.int32),
            pltpu.VMEM((2, BPW, D), jnp.float32),
            pltpu.VMEM((MAX_LEN, D), jnp.float32),
            pltpu.SemaphoreType.DMA,
            pltpu.SemaphoreType.DMA,
            pltpu.SemaphoreType.DMA,
            pltpu.SemaphoreType.DMA,
            pltpu.SemaphoreType.DMA,
            pltpu.SemaphoreType.DMA,
        ],
        compiler_params=pltpu.CompilerParams(use_tc_tiling_on_sc=False),
    )
    def emb_kernel(xt_hbm, tok_hbm, pos_hbm, out_hbm, idx_v, rows_v, pos_v,
                   isem0, isem1, gsem0, gsem1, osem0, osem1):
        wid = lax.axis_index("s") * NC + lax.axis_index("c")
        b0 = wid * BPW
        isems = (isem0, isem1)
        gsems = (gsem0, gsem1)
        osems = (osem0, osem1)

        pltpu.sync_copy(pos_hbm, pos_v)

        def idx_copy(l, slot):
            return pltpu.make_async_copy(
                xt_hbm.at[l, pl.ds(b0, BPW)], idx_v.at[slot], isems[slot]
            )

        def gather_copy(slot):
            return pltpu.make_async_copy(
                tok_hbm.at[idx_v.at[slot]], rows_v.at[slot], gsems[slot]
            )

        def out_copy(l, slot):
            return pltpu.make_async_copy(
                rows_v.at[slot],
                out_hbm.at[pl.ds(b0, BPW), l, pl.ds(0, D)],
                osems[slot],
            )

        def add_pos(l, slot):
            pvs = [pos_v[l, pl.ds(j * 16, 16)] for j in range(D // 16)]

            def body(t, c):
                for j in range(D // 16):
                    plsc.addupdate(rows_v.at[slot, t, pl.ds(j * 16, 16)], pvs[j])
                return c

            lax.fori_loop(0, BPW, body, 0)

        idx_copy(0, 0).start()
        idx_copy(1, 1).start()
        idx_copy(0, 0).wait()
        gather_copy(0).start()

        def step(l, slot, guard_tail):
            other = 1 - slot
            gather_copy(slot).wait()
            if guard_tail:
                @pl.when(l + 2 < SEQ)
                def _():
                    idx_copy(l + 2, slot).start()

                @pl.when(l + 1 < SEQ)
                def _():
                    idx_copy(l + 1, other).wait()
                    @pl.when(l - 1 >= 0)
                    def _():
                        out_copy(l - 1, other).wait()
                    gather_copy(other).start()
            else:
                idx_copy(l + 2, slot).start()
                idx_copy(l + 1, other).wait()
                @pl.when(l - 1 >= 0)
                def _():
                    out_copy(l - 1, other).wait()
                gather_copy(other).start()
            add_pos(l, slot)
            out_copy(l, slot).start()

        def body(g, carry):
            l0 = g * 2
            guard = True
            step(l0, 0, guard)
            step(l0 + 1, 1, guard)
            return carry

        lax.fori_loop(0, SEQ // 2, body, 0)

        out_copy(SEQ - 2, 0).wait()
        out_copy(SEQ - 1, 1).wait()

    return emb_kernel


_emb = _make_kernel()


@jax.jit
def kernel(x, token_table, pos_table):
    xt = x.astype(jnp.int32).T
    out = _emb(xt, token_table, pos_table)
    return out[:, :, :D]


# speedup vs baseline: 1.4658x; 1.0703x over previous
import functools

import jax
import jax.numpy as jnp
from jax import lax
from jax.experimental import pallas as pl
from jax.experimental.pallas import tpu as pltpu
from jax.experimental.pallas import tpu_sc as plsc

MAX_LEN = 200
D = 64
DPAD = 128
BATCH = 4096
SEQ = 200

NC = 2
NS = 16
NW = NC * NS
BPW = BATCH // NW


def _make_kernel():
    mesh = plsc.VectorSubcoreMesh(core_axis_name="c", subcore_axis_name="s")

    @functools.partial(
        pl.kernel,
        mesh=mesh,
        out_type=jax.ShapeDtypeStruct((BATCH, SEQ, DPAD), jnp.float32),
        scratch_types=[
            pltpu.VMEM((2, BPW), jnp.int32),
            pltpu.VMEM((2, BPW), jnp.int32),
            pltpu.VMEM((2, BPW, D), jnp.float32),
            pltpu.VMEM((MAX_LEN, D), jnp.float32),
            pltpu.SemaphoreType.DMA,
            pltpu.SemaphoreType.DMA,
            pltpu.SemaphoreType.DMA,
            pltpu.SemaphoreType.DMA,
            pltpu.SemaphoreType.DMA,
            pltpu.SemaphoreType.DMA,
        ],
        compiler_params=pltpu.CompilerParams(use_tc_tiling_on_sc=False),
    )
    def emb_kernel(xt_hbm, tok_hbm, pos_hbm, out_hbm, idx_v, idx2_v, rows_v,
                   pos_v, isem0, isem1, gsem0, gsem1, osem0, osem1):
        wid = lax.axis_index("s") * NC + lax.axis_index("c")
        b0 = wid * BPW
        isems = (isem0, isem1)
        gsems = (gsem0, gsem1)
        osems = (osem0, osem1)

        pltpu.sync_copy(pos_hbm, pos_v)

        def idx_copy(l, slot):
            return pltpu.make_async_copy(
                xt_hbm.at[l, pl.ds(b0, BPW)], idx_v.at[slot], isems[slot]
            )

        def gather_copy(slot):
            return pltpu.make_async_copy(
                tok_hbm.at[idx2_v.at[slot]], rows_v.at[slot], gsems[slot]
            )

        def double_idx(slot):
            for j in range(BPW // 16):
                sl = pl.ds(j * 16, 16)
                idx2_v[slot, sl] = idx_v[slot, sl] * 2

        def out_copy(l, slot):
            return pltpu.make_async_copy(
                rows_v.at[slot],
                out_hbm.at[pl.ds(b0, BPW), l, pl.ds(0, D)],
                osems[slot],
            )

        def add_pos(l, slot):
            pvs = [pos_v[l, pl.ds(j * 16, 16)] for j in range(D // 16)]

            def body(t, c):
                for j in range(D // 16):
                    plsc.addupdate(rows_v.at[slot, t, pl.ds(j * 16, 16)], pvs[j])
                return c

            lax.fori_loop(0, BPW, body, 0)

        idx_copy(0, 0).start()
        idx_copy(1, 1).start()
        idx_copy(0, 0).wait()
        double_idx(0)
        gather_copy(0).start()

        def step(l, slot, guard_tail):
            other = 1 - slot
            gather_copy(slot).wait()
            if guard_tail:
                @pl.when(l + 2 < SEQ)
                def _():
                    idx_copy(l + 2, slot).start()

                @pl.when(l + 1 < SEQ)
                def _():
                    idx_copy(l + 1, other).wait()
                    double_idx(other)
                    @pl.when(l - 1 >= 0)
                    def _():
                        out_copy(l - 1, other).wait()
                    gather_copy(other).start()
            else:
                idx_copy(l + 2, slot).start()
                idx_copy(l + 1, other).wait()
                double_idx(other)
                @pl.when(l - 1 >= 0)
                def _():
                    out_copy(l - 1, other).wait()
                gather_copy(other).start()
            add_pos(l, slot)
            out_copy(l, slot).start()

        def body(g, carry):
            l0 = g * 2
            guard = True
            step(l0, 0, guard)
            step(l0 + 1, 1, guard)
            return carry

        lax.fori_loop(0, SEQ // 2, body, 0)

        out_copy(SEQ - 2, 0).wait()
        out_copy(SEQ - 1, 1).wait()

    return emb_kernel


_emb = _make_kernel()


@jax.jit
def kernel(x, token_table, pos_table):
    xt = x.astype(jnp.int32).T
    tpad = jnp.pad(token_table, ((0, 0), (0, DPAD - D))).reshape(-1, D)
    out = _emb(xt, tpad, pos_table)
    return out[:, :, :D]
